# baseline (speedup 1.0000x reference)
import jax
import jax.numpy as jnp
from jax import lax
from jax.experimental import pallas as pl
from jax.experimental.pallas import tpu as pltpu
from jax.experimental.pallas import tpu_sc as plsc

N = 10000
E = 320000
D_IN = 128
H = 8
O = 16
D = H * O
HID = 128
M = 4

NC = 2
NS = 16
C = 64
CHUNKS = 314
E_PAD = CHUNKS * NS * C
N_ACC = 10080
ROW = D + 16
BN = 1000


def _pre_body(x_ref, w_ref, al_ref, ar_ref, feat_ref, el_ref, er_ref):
    x = x_ref[0]
    f = jnp.dot(x, w_ref[0], preferred_element_type=jnp.float32,
                precision=lax.Precision.HIGHEST)
    feat_ref[0] = f
    el_ref[0] = jnp.dot(f, al_ref[0], preferred_element_type=jnp.float32,
                        precision=lax.Precision.HIGHEST)
    er_ref[0] = jnp.dot(f, ar_ref[0], preferred_element_type=jnp.float32,
                        precision=lax.Precision.HIGHEST)


def _pre(xs, Ws, Al16, Ar16):
    nb = N // BN
    return pl.pallas_call(
        _pre_body,
        grid=(M, nb),
        in_specs=[
            pl.BlockSpec((1, BN, D_IN), lambda m, i: (m, i, 0)),
            pl.BlockSpec((1, D_IN, D), lambda m, i: (m, 0, 0)),
            pl.BlockSpec((1, D, 16), lambda m, i: (m, 0, 0)),
            pl.BlockSpec((1, D, 16), lambda m, i: (m, 0, 0)),
        ],
        out_specs=[
            pl.BlockSpec((1, BN, D), lambda m, i: (m, i, 0)),
            pl.BlockSpec((1, BN, 16), lambda m, i: (m, i, 0)),
            pl.BlockSpec((1, BN, 16), lambda m, i: (m, i, 0)),
        ],
        out_shape=[
            jax.ShapeDtypeStruct((M, N, D), jnp.float32),
            jax.ShapeDtypeStruct((M, N, 16), jnp.float32),
            jax.ShapeDtypeStruct((M, N, 16), jnp.float32),
        ],
    )(xs, Ws, Al16, Ar16)


def _edge_body(featc, el16, er16, idx3, nums,
               idx_v0, a_v0, b_v0, feat_v0, out_v0,
               dstl_s0, semi0, semg0, sems0,
               idx_v1, a_v1, b_v1, feat_v1, out_v1,
               dstl_s1, semi1, semg1, sems1, acc):
    c = lax.axis_index("c")
    s = lax.axis_index("s")
    lane = lax.iota(jnp.int32, 16)
    headmask = lane < H
    zero16 = jnp.zeros((16,), jnp.float32)
    bufs = ((idx_v0, a_v0, b_v0, feat_v0, out_v0,
             dstl_s0, semi0, semg0, sems0),
            (idx_v1, a_v1, b_v1, feat_v1, out_v1,
             dstl_s1, semi1, semg1, sems1))

    def start(m, chunk, buf):
        idx_v, a_v, b_v, feat_v, _, _, semi, semg, _ = buf
        pltpu.async_copy(idx3.at[m, chunk], idx_v, semi).wait()
        pltpu.async_copy(featc.at[idx_v.at[0]], feat_v, semg)
        pltpu.async_copy(el16.at[idx_v.at[0]], a_v, semg)
        pltpu.async_copy(er16.at[idx_v.at[1]], b_v, semg)

    def wait_gathers(buf):
        idx_v, a_v, b_v, feat_v, _, _, _, semg, _ = buf
        pltpu.make_async_copy(featc.at[idx_v.at[0]], feat_v, semg).wait()
        pltpu.make_async_copy(el16.at[idx_v.at[0]], a_v, semg).wait()
        pltpu.make_async_copy(er16.at[idx_v.at[1]], b_v, semg).wait()

    def wait_scatter(buf):
        _, _, _, _, out_v, dstl_s, _, _, sems = buf
        pltpu.make_async_copy(out_v, acc.at[dstl_s], sems).wait()

    def compute(buf):
        _, a_v, b_v, feat_v, out_v, _, _, _, _ = buf

        def _edge(e, _):
            sv = a_v[e] + b_v[e]
            act = jnp.maximum(sv, 0.2 * sv)
            ex = jnp.exp(act)
            out_v[e, pl.ds(D, 16)] = jnp.where(headmask, ex, 0.0)
            for h in range(H):
                mh = lax.gather(
                    ex, jnp.full((16, 1), h, jnp.int32),
                    lax.GatherDimensionNumbers(
                        offset_dims=(), collapsed_slice_dims=(0,),
                        start_index_map=(0,)),
                    (1,), mode=lax.GatherScatterMode.PROMISE_IN_BOUNDS)
                out_v[e, pl.ds(h * O, O)] = feat_v[e, pl.ds(h * O, O)] * mh
            return 0
        lax.fori_loop(0, C, _edge, 0, unroll=4)

    def scatter(buf):
        idx_v, _, _, _, out_v, dstl_s, _, _, sems = buf
        for k in range(C // 16):
            dstl_s[pl.ds(k * 16, 16)] = idx_v[2, pl.ds(k * 16, 16)]
        pltpu.async_copy(out_v, acc.at[dstl_s], sems, add=True)

    PAIRS = CHUNKS // 2
    for mi in range(2):
        m = 2 * c + mi

        def _z(i, _):
            for k in range(ROW // 16):
                out_v0[i, pl.ds(k * 16, 16)] = zero16
            return 0
        lax.fori_loop(0, C, _z, 0)
        for k in range(9):
            pltpu.sync_copy(out_v0.at[pl.ds(0, C)],
                            acc.at[pl.ds(s * (N_ACC // NS) + k * C, C)])
        pltpu.sync_copy(out_v0.at[pl.ds(0, 54)],
                        acc.at[pl.ds(s * (N_ACC // NS) + 9 * C, 54)])
        plsc.subcore_barrier()

        tile_base = s * CHUNKS
        start(m, tile_base, bufs[0])

        def _pair(i, _):
            g0 = tile_base + 2 * i
            start(m, g0 + 1, bufs[1])
            wait_gathers(bufs[0])

            @pl.when(i > 0)
            def _():
                wait_scatter(bufs[0])
            compute(bufs[0])
            scatter(bufs[0])

            @pl.when(i + 1 < PAIRS)
            def _():
                start(m, g0 + 2, bufs[0])
            wait_gathers(bufs[1])

            @pl.when(i > 0)
            def _():
                wait_scatter(bufs[1])
            compute(bufs[1])
            scatter(bufs[1])
            return 0
        lax.fori_loop(0, PAIRS, _pair, 0)
        wait_scatter(bufs[0])
        wait_scatter(bufs[1])
        plsc.subcore_barrier()

        pltpu.sync_copy(acc.at[pl.ds(s * (N_ACC // NS), N_ACC // NS)],
                        nums.at[m, pl.ds(s * (N_ACC // NS), N_ACC // NS)])
        plsc.subcore_barrier()


def _edge_phase(featc, el16, er16, idx3):
    mesh = plsc.VectorSubcoreMesh(core_axis_name="c", subcore_axis_name="s")
    return pl.kernel(
        _edge_body,
        out_type=jax.ShapeDtypeStruct((M, N_ACC, ROW), jnp.float32),
        mesh=mesh,
        scratch_types=[
            pltpu.VMEM((3, C), jnp.int32),
            pltpu.VMEM((C, 16), jnp.float32),
            pltpu.VMEM((C, 16), jnp.float32),
            pltpu.VMEM((C, D), jnp.float32),
            pltpu.VMEM((C, ROW), jnp.float32),
            pltpu.VMEM((C,), jnp.int32),
            pltpu.SemaphoreType.DMA,
            pltpu.SemaphoreType.DMA,
            pltpu.SemaphoreType.DMA,
            pltpu.VMEM((3, C), jnp.int32),
            pltpu.VMEM((C, 16), jnp.float32),
            pltpu.VMEM((C, 16), jnp.float32),
            pltpu.VMEM((C, D), jnp.float32),
            pltpu.VMEM((C, ROW), jnp.float32),
            pltpu.VMEM((C,), jnp.int32),
            pltpu.SemaphoreType.DMA,
            pltpu.SemaphoreType.DMA,
            pltpu.SemaphoreType.DMA,
            pltpu.VMEM_SHARED((N_ACC, ROW), jnp.float32),
        ],
        compiler_params=pltpu.CompilerParams(use_tc_tiling_on_sc=False),
    )(featc, el16, er16, idx3)


def _post_body(nums_ref, r16_ref, bs_ref, p1w_ref, p1b_ref,
               gat_ref, s_ref):
    nb = pl.program_id(1)
    blk = nums_ref[0]
    num = blk[:, :D]
    den = jnp.dot(blk[:, D:ROW], r16_ref[...],
                  preferred_element_type=jnp.float32,
                  precision=lax.Precision.HIGHEST)
    g = num / jnp.maximum(den, 1e-16) + bs_ref[0][:1, :]
    g = jnp.maximum(g, 0.01 * g)
    gat_ref[0] = g
    th = jnp.tanh(jnp.dot(g, p1w_ref[...], preferred_element_type=jnp.float32,
                          precision=lax.Precision.HIGHEST) + p1b_ref[...])
    colsum = jnp.sum(th, axis=0, keepdims=True)

    @pl.when(nb == 0)
    def _():
        s_ref[...] = jnp.zeros_like(s_ref)
    s_ref[0] = s_ref[0] + colsum


def _post(nums, R16, bs, P1w, P1b):
    nb = N // BN
    return pl.pallas_call(
        _post_body,
        grid=(M, nb),
        in_specs=[
            pl.BlockSpec((1, BN, ROW), lambda m, i: (m, i, 0)),
            pl.BlockSpec((16, D), lambda m, i: (0, 0)),
            pl.BlockSpec((1, 8, D), lambda m, i: (m, 0, 0)),
            pl.BlockSpec((D, HID), lambda m, i: (0, 0)),
            pl.BlockSpec((1, HID), lambda m, i: (0, 0)),
        ],
        out_specs=[
            pl.BlockSpec((1, BN, D), lambda m, i: (m, i, 0)),
            pl.BlockSpec((1, 8, HID), lambda m, i: (m, 0, 0)),
        ],
        out_shape=[
            jax.ShapeDtypeStruct((M, N, D), jnp.float32),
            jax.ShapeDtypeStruct((M, 8, HID), jnp.float32),
        ],
    )(nums, R16, bs, P1w, P1b)


def _blend_body(gat_ref, s_ref, p2_ref, lnc_ref, dis_ref):
    w = jnp.dot(s_ref[...], p2_ref[...], preferred_element_type=jnp.float32,
                precision=lax.Precision.HIGHEST) / N

    def beta(i, j):
        wi = w[i, 0]
        wj = w[j, 0]
        mx = jnp.maximum(wi, wj)
        ei = jnp.exp(wi - mx)
        ej = jnp.exp(wj - mx)
        return ei / (ei + ej)

    b12 = beta(1, 2)
    lnc_ref[...] = b12 * gat_ref[1] + (1.0 - b12) * gat_ref[2]
    b03 = beta(0, 3)
    dis_ref[...] = b03 * gat_ref[0] + (1.0 - b03) * gat_ref[3]


def _blend(gat, s, P2w):
    nb = N // BN
    return pl.pallas_call(
        _blend_body,
        grid=(nb,),
        in_specs=[
            pl.BlockSpec((M, BN, D), lambda i: (0, i, 0)),
            pl.BlockSpec((M, HID), lambda i: (0, 0)),
            pl.BlockSpec((HID, 1), lambda i: (0, 0)),
        ],
        out_specs=[
            pl.BlockSpec((BN, D), lambda i: (i, 0)),
            pl.BlockSpec((BN, D), lambda i: (i, 0)),
        ],
        out_shape=[
            jax.ShapeDtypeStruct((N, D), jnp.float32),
            jax.ShapeDtypeStruct((N, D), jnp.float32),
        ],
    )(gat, s, P2w)


def kernel(x0, x1, x2, x3, edge_index0, edge_index1, edge_index2, edge_index3,
           W0, W1, W2, W3, al0, al1, al2, al3, ar0, ar1, ar2, ar3,
           b0, b1, b2, b3, P1w, P1b, P2w):
    xs = jnp.stack([x0, x1, x2, x3])
    Ws = jnp.stack([W0, W1, W2, W3])
    bs = jnp.stack([b0, b1, b2, b3])

    rows = jnp.arange(D)
    head_of = rows // O

    def blockdiag(a):
        return jnp.zeros((D, 16), jnp.float32).at[rows, head_of].set(
            a.reshape(D))

    Al16 = jnp.stack([blockdiag(a) for a in (al0, al1, al2, al3)])
    Ar16 = jnp.stack([blockdiag(a) for a in (ar0, ar1, ar2, ar3)])
    R16 = jnp.zeros((16, D), jnp.float32).at[head_of, rows].set(1.0)

    feat, el, er = _pre(xs, Ws, Al16, Ar16)
    featc = feat.reshape(M * N, D)
    el16 = el.reshape(M * N, 16)
    er16 = er.reshape(M * N, 16)

    pad = E_PAD - E
    eis = jnp.stack([edge_index0, edge_index1, edge_index2, edge_index3])
    offs = (jnp.arange(M, dtype=jnp.int32) * N)[:, None]
    srcg = jnp.concatenate(
        [eis[:, 0, :] + offs, jnp.broadcast_to(offs, (M, pad))], axis=1)
    dstg = jnp.concatenate(
        [eis[:, 1, :] + offs, jnp.broadcast_to(offs, (M, pad))], axis=1)
    dstl = jnp.concatenate(
        [eis[:, 1, :], jnp.full((M, pad), N, jnp.int32)], axis=1)
    nch = E_PAD // C
    idx3 = jnp.stack([srcg.reshape(M, nch, C), dstg.reshape(M, nch, C),
                      dstl.reshape(M, nch, C)], axis=2)

    nums = _edge_phase(featc, el16, er16, idx3)

    bs8 = jnp.broadcast_to(bs[:, None, :], (M, 8, D))
    gat, s8 = _post(nums, R16, bs8, P1w, P1b.reshape(1, HID))
    lnc, dis = _blend(gat, s8[:, 0, :], P2w)
    return (lnc, dis)


# speedup vs baseline: 1.9390x; 1.9390x over previous
import jax
import jax.numpy as jnp
from jax import lax
from jax.experimental import pallas as pl
from jax.experimental.pallas import tpu as pltpu
from jax.experimental.pallas import tpu_sc as plsc

N = 10000
E = 320000
D_IN = 128
H = 8
O = 16
D = H * O
HID = 128
M = 4

NC = 2
NS = 16
C = 64
CHUNKS = 314
E_PAD = CHUNKS * NS * C
N_ACC = 10080
ROW = D + 16
BN = 1000


def _pre_body(x_ref, w_ref, al_ref, ar_ref, p_ref, feat_ref, el_ref, er_ref):
    x = x_ref[0]
    f = jnp.dot(x, w_ref[0], preferred_element_type=jnp.float32,
                precision=lax.Precision.HIGHEST)
    feat_ref[0] = jnp.dot(f, p_ref[...], preferred_element_type=jnp.float32,
                          precision=lax.Precision.HIGHEST)
    el_ref[0] = jnp.dot(f, al_ref[0], preferred_element_type=jnp.float32,
                        precision=lax.Precision.HIGHEST)
    er_ref[0] = jnp.dot(f, ar_ref[0], preferred_element_type=jnp.float32,
                        precision=lax.Precision.HIGHEST)


def _pre(xs, Ws, Al16, Ar16, P):
    nb = N // BN
    return pl.pallas_call(
        _pre_body,
        grid=(M, nb),
        in_specs=[
            pl.BlockSpec((1, BN, D_IN), lambda m, i: (m, i, 0)),
            pl.BlockSpec((1, D_IN, D), lambda m, i: (m, 0, 0)),
            pl.BlockSpec((1, D, 16), lambda m, i: (m, 0, 0)),
            pl.BlockSpec((1, D, 16), lambda m, i: (m, 0, 0)),
            pl.BlockSpec((D, D), lambda m, i: (0, 0)),
        ],
        out_specs=[
            pl.BlockSpec((1, BN, D), lambda m, i: (m, i, 0)),
            pl.BlockSpec((1, BN, 16), lambda m, i: (m, i, 0)),
            pl.BlockSpec((1, BN, 16), lambda m, i: (m, i, 0)),
        ],
        out_shape=[
            jax.ShapeDtypeStruct((M, N, D), jnp.float32),
            jax.ShapeDtypeStruct((M, N, 16), jnp.float32),
            jax.ShapeDtypeStruct((M, N, 16), jnp.float32),
        ],
    )(xs, Ws, Al16, Ar16, P)


def _edge_body(featc, el16, er16, idx3, nums,
               idx_v0, a_v0, b_v0, feat_v0, out_v0,
               dstl_s0, semi0, semg0, sems0,
               idx_v1, a_v1, b_v1, feat_v1, out_v1,
               dstl_s1, semi1, semg1, sems1, acc):
    c = lax.axis_index("c")
    s = lax.axis_index("s")
    lane = lax.iota(jnp.int32, 16)
    headmask = lane < H
    zero16 = jnp.zeros((16,), jnp.float32)
    bufs = ((idx_v0, a_v0, b_v0, feat_v0, out_v0,
             dstl_s0, semi0, semg0, sems0),
            (idx_v1, a_v1, b_v1, feat_v1, out_v1,
             dstl_s1, semi1, semg1, sems1))

    def start(m, chunk, buf):
        idx_v, a_v, b_v, feat_v, _, _, semi, semg, _ = buf
        pltpu.async_copy(idx3.at[m, chunk], idx_v, semi).wait()
        pltpu.async_copy(featc.at[idx_v.at[0]], feat_v, semg)
        pltpu.async_copy(el16.at[idx_v.at[0]], a_v, semg)
        pltpu.async_copy(er16.at[idx_v.at[1]], b_v, semg)

    def wait_gathers(buf):
        idx_v, a_v, b_v, feat_v, _, _, _, semg, _ = buf
        pltpu.make_async_copy(featc.at[idx_v.at[0]], feat_v, semg).wait()
        pltpu.make_async_copy(el16.at[idx_v.at[0]], a_v, semg).wait()
        pltpu.make_async_copy(er16.at[idx_v.at[1]], b_v, semg).wait()

    def wait_scatter(buf):
        _, _, _, _, out_v, dstl_s, _, _, sems = buf
        pltpu.make_async_copy(out_v, acc.at[dstl_s], sems).wait()

    def compute(buf):
        _, a_v, b_v, feat_v, out_v, _, _, _, _ = buf

        @plsc.parallel_loop(0, C, 1, unroll=4)
        def _edge(e):
            sv = a_v[e] + b_v[e]
            act = jnp.maximum(sv, 0.2 * sv)
            exx = jnp.exp(act)
            out_v[e, pl.ds(D, 16)] = jnp.where(headmask, exx, 0.0)
            for g in range(H):
                out_v[e, pl.ds(g * O, O)] = feat_v[e, pl.ds(g * O, O)] * exx

    def scatter(buf):
        idx_v, _, _, _, out_v, dstl_s, _, _, sems = buf
        for k in range(C // 16):
            dstl_s[pl.ds(k * 16, 16)] = idx_v[2, pl.ds(k * 16, 16)]
        pltpu.async_copy(out_v, acc.at[dstl_s], sems, add=True)

    PAIRS = CHUNKS // 2
    for mi in range(2):
        m = 2 * c + mi

        def _z(i, _):
            for k in range(ROW // 16):
                out_v0[i, pl.ds(k * 16, 16)] = zero16
            return 0
        lax.fori_loop(0, C, _z, 0)
        for k in range(9):
            pltpu.sync_copy(out_v0.at[pl.ds(0, C)],
                            acc.at[pl.ds(s * (N_ACC // NS) + k * C, C)])
        pltpu.sync_copy(out_v0.at[pl.ds(0, 54)],
                        acc.at[pl.ds(s * (N_ACC // NS) + 9 * C, 54)])
        plsc.subcore_barrier()

        tile_base = s * CHUNKS
        start(m, tile_base, bufs[0])

        def _pair(i, _):
            g0 = tile_base + 2 * i
            start(m, g0 + 1, bufs[1])
            wait_gathers(bufs[0])

            @pl.when(i > 0)
            def _():
                wait_scatter(bufs[0])
            compute(bufs[0])
            scatter(bufs[0])

            @pl.when(i + 1 < PAIRS)
            def _():
                start(m, g0 + 2, bufs[0])
            wait_gathers(bufs[1])

            @pl.when(i > 0)
            def _():
                wait_scatter(bufs[1])
            compute(bufs[1])
            scatter(bufs[1])
            return 0
        lax.fori_loop(0, PAIRS, _pair, 0)
        wait_scatter(bufs[0])
        wait_scatter(bufs[1])
        plsc.subcore_barrier()

        pltpu.sync_copy(acc.at[pl.ds(s * (N_ACC // NS), N_ACC // NS)],
                        nums.at[m, pl.ds(s * (N_ACC // NS), N_ACC // NS)])
        plsc.subcore_barrier()


def _edge_phase(featc, el16, er16, idx3):
    mesh = plsc.VectorSubcoreMesh(core_axis_name="c", subcore_axis_name="s")
    return pl.kernel(
        _edge_body,
        out_type=jax.ShapeDtypeStruct((M, N_ACC, ROW), jnp.float32),
        mesh=mesh,
        scratch_types=[
            pltpu.VMEM((3, C), jnp.int32),
            pltpu.VMEM((C, 16), jnp.float32),
            pltpu.VMEM((C, 16), jnp.float32),
            pltpu.VMEM((C, D), jnp.float32),
            pltpu.VMEM((C, ROW), jnp.float32),
            pltpu.VMEM((C,), jnp.int32),
            pltpu.SemaphoreType.DMA,
            pltpu.SemaphoreType.DMA,
            pltpu.SemaphoreType.DMA,
            pltpu.VMEM((3, C), jnp.int32),
            pltpu.VMEM((C, 16), jnp.float32),
            pltpu.VMEM((C, 16), jnp.float32),
            pltpu.VMEM((C, D), jnp.float32),
            pltpu.VMEM((C, ROW), jnp.float32),
            pltpu.VMEM((C,), jnp.int32),
            pltpu.SemaphoreType.DMA,
            pltpu.SemaphoreType.DMA,
            pltpu.SemaphoreType.DMA,
            pltpu.VMEM_SHARED((N_ACC, ROW), jnp.float32),
        ],
        compiler_params=pltpu.CompilerParams(use_tc_tiling_on_sc=False),
    )(featc, el16, er16, idx3)


def _post_body(nums_ref, r16_ref, pt_ref, bs_ref, p1w_ref, p1b_ref,
               gat_ref, s_ref):
    nb = pl.program_id(1)
    blk = nums_ref[0]
    num = blk[:, :D]
    den = jnp.dot(blk[:, D:ROW], r16_ref[...],
                  preferred_element_type=jnp.float32,
                  precision=lax.Precision.HIGHEST)
    t = num / jnp.maximum(den, 1e-16)
    g = jnp.dot(t, pt_ref[...], preferred_element_type=jnp.float32,
                precision=lax.Precision.HIGHEST) + bs_ref[0][:1, :]
    g = jnp.maximum(g, 0.01 * g)
    gat_ref[0] = g
    th = jnp.tanh(jnp.dot(g, p1w_ref[...], preferred_element_type=jnp.float32,
                          precision=lax.Precision.HIGHEST) + p1b_ref[...])
    colsum = jnp.sum(th, axis=0, keepdims=True)

    @pl.when(nb == 0)
    def _():
        s_ref[...] = jnp.zeros_like(s_ref)
    s_ref[0] = s_ref[0] + colsum


def _post(nums, R16, Pt, bs, P1w, P1b):
    nb = N // BN
    return pl.pallas_call(
        _post_body,
        grid=(M, nb),
        in_specs=[
            pl.BlockSpec((1, BN, ROW), lambda m, i: (m, i, 0)),
            pl.BlockSpec((16, D), lambda m, i: (0, 0)),
            pl.BlockSpec((D, D), lambda m, i: (0, 0)),
            pl.BlockSpec((1, 8, D), lambda m, i: (m, 0, 0)),
            pl.BlockSpec((D, HID), lambda m, i: (0, 0)),
            pl.BlockSpec((1, HID), lambda m, i: (0, 0)),
        ],
        out_specs=[
            pl.BlockSpec((1, BN, D), lambda m, i: (m, i, 0)),
            pl.BlockSpec((1, 8, HID), lambda m, i: (m, 0, 0)),
        ],
        out_shape=[
            jax.ShapeDtypeStruct((M, N, D), jnp.float32),
            jax.ShapeDtypeStruct((M, 8, HID), jnp.float32),
        ],
    )(nums, R16, Pt, bs, P1w, P1b)


def _blend_body(gat_ref, s_ref, p2_ref, lnc_ref, dis_ref):
    w = jnp.dot(s_ref[...], p2_ref[...], preferred_element_type=jnp.float32,
                precision=lax.Precision.HIGHEST) / N

    def beta(i, j):
        wi = w[i, 0]
        wj = w[j, 0]
        mx = jnp.maximum(wi, wj)
        ei = jnp.exp(wi - mx)
        ej = jnp.exp(wj - mx)
        return ei / (ei + ej)

    b12 = beta(1, 2)
    lnc_ref[...] = b12 * gat_ref[1] + (1.0 - b12) * gat_ref[2]
    b03 = beta(0, 3)
    dis_ref[...] = b03 * gat_ref[0] + (1.0 - b03) * gat_ref[3]


def _blend(gat, s, P2w):
    nb = N // BN
    return pl.pallas_call(
        _blend_body,
        grid=(nb,),
        in_specs=[
            pl.BlockSpec((M, BN, D), lambda i: (0, i, 0)),
            pl.BlockSpec((M, HID), lambda i: (0, 0)),
            pl.BlockSpec((HID, 1), lambda i: (0, 0)),
        ],
        out_specs=[
            pl.BlockSpec((BN, D), lambda i: (i, 0)),
            pl.BlockSpec((BN, D), lambda i: (i, 0)),
        ],
        out_shape=[
            jax.ShapeDtypeStruct((N, D), jnp.float32),
            jax.ShapeDtypeStruct((N, D), jnp.float32),
        ],
    )(gat, s, P2w)


def kernel(x0, x1, x2, x3, edge_index0, edge_index1, edge_index2, edge_index3,
           W0, W1, W2, W3, al0, al1, al2, al3, ar0, ar1, ar2, ar3,
           b0, b1, b2, b3, P1w, P1b, P2w):
    xs = jnp.stack([x0, x1, x2, x3])
    Ws = jnp.stack([W0, W1, W2, W3])
    bs = jnp.stack([b0, b1, b2, b3])

    rows = jnp.arange(D)
    head_of = rows // O

    def blockdiag(a):
        z = jnp.zeros((D, 16), jnp.float32).at[rows, head_of].set(a.reshape(D))
        return z.at[rows, head_of + 8].set(a.reshape(D))

    Al16 = jnp.stack([blockdiag(a) for a in (al0, al1, al2, al3)])
    Ar16 = jnp.stack([blockdiag(a) for a in (ar0, ar1, ar2, ar3)])
    P = jnp.zeros((D, D), jnp.float32).at[rows, (rows % O) * H + head_of].set(
        1.0)
    Pt = P.T
    R16p = jnp.zeros((16, D), jnp.float32).at[rows % H, rows].set(1.0)

    feat, el, er = _pre(xs, Ws, Al16, Ar16, P)
    featc = feat.reshape(M * N, D)
    el16 = el.reshape(M * N, 16)
    er16 = er.reshape(M * N, 16)

    pad = E_PAD - E
    eis = jnp.stack([edge_index0, edge_index1, edge_index2, edge_index3])
    offs = (jnp.arange(M, dtype=jnp.int32) * N)[:, None]
    srcg = jnp.concatenate(
        [eis[:, 0, :] + offs, jnp.broadcast_to(offs, (M, pad))], axis=1)
    dstg = jnp.concatenate(
        [eis[:, 1, :] + offs, jnp.broadcast_to(offs, (M, pad))], axis=1)
    dstl = jnp.concatenate(
        [eis[:, 1, :], jnp.full((M, pad), N, jnp.int32)], axis=1)
    nch = E_PAD // C
    idx3 = jnp.stack([srcg.reshape(M, nch, C), dstg.reshape(M, nch, C),
                      dstl.reshape(M, nch, C)], axis=2)

    nums = _edge_phase(featc, el16, er16, idx3)

    bs8 = jnp.broadcast_to(bs[:, None, :], (M, 8, D))
    gat, s8 = _post(nums, R16p, Pt, bs8, P1w, P1b.reshape(1, HID))
    lnc, dis = _blend(gat, s8[:, 0, :], P2w)
    return (lnc, dis)


# speedup vs baseline: 2.1216x; 1.0942x over previous
import jax
import jax.numpy as jnp
from jax import lax
from jax.experimental import pallas as pl
from jax.experimental.pallas import tpu as pltpu
from jax.experimental.pallas import tpu_sc as plsc

N = 10000
E = 320000
D_IN = 128
H = 8
O = 16
D = H * O
HID = 128
M = 4

NC = 2
NS = 16
C = 64
CHUNKS = 314
E_PAD = CHUNKS * NS * C
N_ACC = 10080
ROW = D + 16
BN = 2000


def _pre_body(x_ref, w_ref, g_ref, ar_ref, feat_ref, er_ref):
    x = x_ref[0]
    f = jnp.dot(x, w_ref[0], preferred_element_type=jnp.float32,
                precision=lax.Precision.HIGHEST)
    feat_ref[0] = jnp.dot(f, g_ref[0], preferred_element_type=jnp.float32,
                          precision=lax.Precision.HIGHEST)
    er_ref[0] = jnp.dot(f, ar_ref[0], preferred_element_type=jnp.float32,
                        precision=lax.Precision.HIGHEST)


def _pre(xs, Ws, G, Ar16):
    nb = N // BN
    return pl.pallas_call(
        _pre_body,
        grid=(M, nb),
        in_specs=[
            pl.BlockSpec((1, BN, D_IN), lambda m, i: (m, i, 0)),
            pl.BlockSpec((1, D_IN, D), lambda m, i: (m, 0, 0)),
            pl.BlockSpec((1, D, ROW), lambda m, i: (m, 0, 0)),
            pl.BlockSpec((1, D, 16), lambda m, i: (m, 0, 0)),
        ],
        out_specs=[
            pl.BlockSpec((1, BN, ROW), lambda m, i: (m, i, 0)),
            pl.BlockSpec((1, BN, 16), lambda m, i: (m, i, 0)),
        ],
        out_shape=[
            jax.ShapeDtypeStruct((M, N, ROW), jnp.float32),
            jax.ShapeDtypeStruct((M, N, 16), jnp.float32),
        ],
    )(xs, Ws, G, Ar16)


def _edge_body(featel, er16, idx3, nums,
               idx_v0, b_v0, feat_v0, out_v0,
               dstl_s0, semi0, semg0, sems0,
               idx_v1, b_v1, feat_v1, out_v1,
               dstl_s1, semi1, semg1, sems1, acc):
    c = lax.axis_index("c")
    s = lax.axis_index("s")
    lane = lax.iota(jnp.int32, 16)
    headmask = lane < H
    zero16 = jnp.zeros((16,), jnp.float32)
    bufs = ((idx_v0, b_v0, feat_v0, out_v0,
             dstl_s0, semi0, semg0, sems0),
            (idx_v1, b_v1, feat_v1, out_v1,
             dstl_s1, semi1, semg1, sems1))

    def start(m, chunk, buf):
        idx_v, b_v, feat_v, _, _, semi, semg, _ = buf
        pltpu.async_copy(idx3.at[m, chunk], idx_v, semi).wait()
        pltpu.async_copy(featel.at[idx_v.at[0]], feat_v, semg)
        pltpu.async_copy(er16.at[idx_v.at[1]], b_v, semg)

    def wait_gathers(buf):
        idx_v, b_v, feat_v, _, _, _, semg, _ = buf
        pltpu.make_async_copy(featel.at[idx_v.at[0]], feat_v, semg).wait()
        pltpu.make_async_copy(er16.at[idx_v.at[1]], b_v, semg).wait()

    def wait_scatter(buf):
        _, _, _, out_v, dstl_s, _, _, sems = buf
        pltpu.make_async_copy(out_v, acc.at[dstl_s], sems).wait()

    def compute(buf):
        _, b_v, feat_v, out_v, _, _, _, _ = buf

        @plsc.parallel_loop(0, C, 1, unroll=4)
        def _edge(e):
            sv = feat_v[e, pl.ds(D, 16)] + b_v[e]
            act = jnp.maximum(sv, 0.2 * sv)
            exx = jnp.exp(act)
            out_v[e, pl.ds(D, 16)] = jnp.where(headmask, exx, 0.0)
            for g in range(H):
                out_v[e, pl.ds(g * O, O)] = feat_v[e, pl.ds(g * O, O)] * exx

    def scatter(buf):
        idx_v, _, _, out_v, dstl_s, _, _, sems = buf
        for k in range(C // 16):
            dstl_s[pl.ds(k * 16, 16)] = idx_v[2, pl.ds(k * 16, 16)]
        pltpu.async_copy(out_v, acc.at[dstl_s], sems, add=True)

    PAIRS = CHUNKS // 2
    for mi in range(2):
        m = 2 * c + mi

        def _z(i, _):
            for k in range(ROW // 16):
                out_v0[i, pl.ds(k * 16, 16)] = zero16
            return 0
        lax.fori_loop(0, C, _z, 0)
        for k in range(9):
            pltpu.sync_copy(out_v0.at[pl.ds(0, C)],
                            acc.at[pl.ds(s * (N_ACC // NS) + k * C, C)])
        pltpu.sync_copy(out_v0.at[pl.ds(0, 54)],
                        acc.at[pl.ds(s * (N_ACC // NS) + 9 * C, 54)])
        plsc.subcore_barrier()

        tile_base = s * CHUNKS
        start(m, tile_base, bufs[0])

        def _pair(i, _):
            g0 = tile_base + 2 * i
            start(m, g0 + 1, bufs[1])
            wait_gathers(bufs[0])

            @pl.when(i > 0)
            def _():
                wait_scatter(bufs[0])
            compute(bufs[0])
            scatter(bufs[0])

            @pl.when(i + 1 < PAIRS)
            def _():
                start(m, g0 + 2, bufs[0])
            wait_gathers(bufs[1])

            @pl.when(i > 0)
            def _():
                wait_scatter(bufs[1])
            compute(bufs[1])
            scatter(bufs[1])
            return 0
        lax.fori_loop(0, PAIRS, _pair, 0)
        wait_scatter(bufs[0])
        wait_scatter(bufs[1])
        plsc.subcore_barrier()

        pltpu.sync_copy(acc.at[pl.ds(s * (N_ACC // NS), N_ACC // NS)],
                        nums.at[m, pl.ds(s * (N_ACC // NS), N_ACC // NS)])
        plsc.subcore_barrier()


def _edge_phase(featel, er16, idx3):
    mesh = plsc.VectorSubcoreMesh(core_axis_name="c", subcore_axis_name="s")
    return pl.kernel(
        _edge_body,
        out_type=jax.ShapeDtypeStruct((M, N_ACC, ROW), jnp.float32),
        mesh=mesh,
        scratch_types=[
            pltpu.VMEM((3, C), jnp.int32),
            pltpu.VMEM((C, 16), jnp.float32),
            pltpu.VMEM((C, ROW), jnp.float32),
            pltpu.VMEM((C, ROW), jnp.float32),
            pltpu.VMEM((C,), jnp.int32),
            pltpu.SemaphoreType.DMA,
            pltpu.SemaphoreType.DMA,
            pltpu.SemaphoreType.DMA,
            pltpu.VMEM((3, C), jnp.int32),
            pltpu.VMEM((C, 16), jnp.float32),
            pltpu.VMEM((C, ROW), jnp.float32),
            pltpu.VMEM((C, ROW), jnp.float32),
            pltpu.VMEM((C,), jnp.int32),
            pltpu.SemaphoreType.DMA,
            pltpu.SemaphoreType.DMA,
            pltpu.SemaphoreType.DMA,
            pltpu.VMEM_SHARED((N_ACC, ROW), jnp.float32),
        ],
        compiler_params=pltpu.CompilerParams(use_tc_tiling_on_sc=False),
    )(featel, er16, idx3)


def _post_body(nums_ref, r16_ref, pt_ref, bs_ref, p1w_ref, p1b_ref,
               gat_ref, s_ref):
    nb = pl.program_id(1)
    blk = nums_ref[0]
    num = blk[:, :D]
    den = jnp.dot(blk[:, D:ROW], r16_ref[...],
                  preferred_element_type=jnp.float32,
                  precision=lax.Precision.HIGHEST)
    t = num / jnp.maximum(den, 1e-16)
    g = jnp.dot(t, pt_ref[...], preferred_element_type=jnp.float32,
                precision=lax.Precision.HIGHEST) + bs_ref[0][:1, :]
    g = jnp.maximum(g, 0.01 * g)
    gat_ref[0] = g
    th = jnp.tanh(jnp.dot(g, p1w_ref[...], preferred_element_type=jnp.float32,
                          precision=lax.Precision.HIGHEST) + p1b_ref[...])
    colsum = jnp.sum(th, axis=0, keepdims=True)

    @pl.when(nb == 0)
    def _():
        s_ref[...] = jnp.zeros_like(s_ref)
    s_ref[0] = s_ref[0] + colsum


def _post(nums, R16, Pt, bs, P1w, P1b):
    nb = N // BN
    return pl.pallas_call(
        _post_body,
        grid=(M, nb),
        in_specs=[
            pl.BlockSpec((1, BN, ROW), lambda m, i: (m, i, 0)),
            pl.BlockSpec((16, D), lambda m, i: (0, 0)),
            pl.BlockSpec((D, D), lambda m, i: (0, 0)),
            pl.BlockSpec((1, 8, D), lambda m, i: (m, 0, 0)),
            pl.BlockSpec((D, HID), lambda m, i: (0, 0)),
            pl.BlockSpec((1, HID), lambda m, i: (0, 0)),
        ],
        out_specs=[
            pl.BlockSpec((1, BN, D), lambda m, i: (m, i, 0)),
            pl.BlockSpec((1, 8, HID), lambda m, i: (m, 0, 0)),
        ],
        out_shape=[
            jax.ShapeDtypeStruct((M, N, D), jnp.float32),
            jax.ShapeDtypeStruct((M, 8, HID), jnp.float32),
        ],
    )(nums, R16, Pt, bs, P1w, P1b)


def _blend_body(gat_ref, s_ref, p2_ref, lnc_ref, dis_ref):
    w = jnp.dot(s_ref[...], p2_ref[...], preferred_element_type=jnp.float32,
                precision=lax.Precision.HIGHEST) / N

    def beta(i, j):
        wi = w[i, 0]
        wj = w[j, 0]
        mx = jnp.maximum(wi, wj)
        ei = jnp.exp(wi - mx)
        ej = jnp.exp(wj - mx)
        return ei / (ei + ej)

    b12 = beta(1, 2)
    lnc_ref[...] = b12 * gat_ref[1] + (1.0 - b12) * gat_ref[2]
    b03 = beta(0, 3)
    dis_ref[...] = b03 * gat_ref[0] + (1.0 - b03) * gat_ref[3]


def _blend(gat, s, P2w):
    nb = N // BN
    return pl.pallas_call(
        _blend_body,
        grid=(nb,),
        in_specs=[
            pl.BlockSpec((M, BN, D), lambda i: (0, i, 0)),
            pl.BlockSpec((M, HID), lambda i: (0, 0)),
            pl.BlockSpec((HID, 1), lambda i: (0, 0)),
        ],
        out_specs=[
            pl.BlockSpec((BN, D), lambda i: (i, 0)),
            pl.BlockSpec((BN, D), lambda i: (i, 0)),
        ],
        out_shape=[
            jax.ShapeDtypeStruct((N, D), jnp.float32),
            jax.ShapeDtypeStruct((N, D), jnp.float32),
        ],
    )(gat, s, P2w)


def kernel(x0, x1, x2, x3, edge_index0, edge_index1, edge_index2, edge_index3,
           W0, W1, W2, W3, al0, al1, al2, al3, ar0, ar1, ar2, ar3,
           b0, b1, b2, b3, P1w, P1b, P2w):
    xs = jnp.stack([x0, x1, x2, x3])
    Ws = jnp.stack([W0, W1, W2, W3])
    bs = jnp.stack([b0, b1, b2, b3])

    rows = jnp.arange(D)
    head_of = rows // O

    def blockdiag(a):
        z = jnp.zeros((D, 16), jnp.float32).at[rows, head_of].set(a.reshape(D))
        return z.at[rows, head_of + 8].set(a.reshape(D))

    Al16 = jnp.stack([blockdiag(a) for a in (al0, al1, al2, al3)])
    Ar16 = jnp.stack([blockdiag(a) for a in (ar0, ar1, ar2, ar3)])
    P = jnp.zeros((D, D), jnp.float32).at[rows, (rows % O) * H + head_of].set(
        1.0)
    Pt = P.T
    R16p = jnp.zeros((16, D), jnp.float32).at[rows % H, rows].set(1.0)
    G = jnp.concatenate([jnp.broadcast_to(P, (M, D, D)), Al16], axis=2)

    feat, er = _pre(xs, Ws, G, Ar16)
    featel = feat.reshape(M * N, ROW)
    er16 = er.reshape(M * N, 16)

    pad = E_PAD - E
    eis = jnp.stack([edge_index0, edge_index1, edge_index2, edge_index3])
    offs = (jnp.arange(M, dtype=jnp.int32) * N)[:, None]
    srcg = jnp.concatenate(
        [eis[:, 0, :] + offs, jnp.broadcast_to(offs, (M, pad))], axis=1)
    dstg = jnp.concatenate(
        [eis[:, 1, :] + offs, jnp.broadcast_to(offs, (M, pad))], axis=1)
    dstl = jnp.concatenate(
        [eis[:, 1, :], jnp.full((M, pad), N, jnp.int32)], axis=1)
    nch = E_PAD // C
    idx3 = jnp.stack([srcg.reshape(M, nch, C), dstg.reshape(M, nch, C),
                      dstl.reshape(M, nch, C)], axis=2)

    nums = _edge_phase(featel, er16, idx3)

    bs8 = jnp.broadcast_to(bs[:, None, :], (M, 8, D))
    gat, s8 = _post(nums, R16p, Pt, bs8, P1w, P1b.reshape(1, HID))
    lnc, dis = _blend(gat, s8[:, 0, :], P2w)
    return (lnc, dis)


# speedup vs baseline: 2.6202x; 1.2350x over previous
import jax
import jax.numpy as jnp
from jax import lax
from jax.experimental import pallas as pl
from jax.experimental.pallas import tpu as pltpu
from jax.experimental.pallas import tpu_sc as plsc

N = 10000
E = 320000
D_IN = 128
H = 8
O = 16
D = H * O
HID = 128
M = 4

NC = 2
NS = 16
C = 64
CHUNKS = 314
E_PAD = CHUNKS * NS * C
N_ACC = 10080
ROW = D + 16
BN = 2000


def _pre_body(x_ref, w_ref, g_ref, ar_ref, feat_ref, er_ref):
    x = x_ref[0]
    f = jnp.dot(x, w_ref[0], preferred_element_type=jnp.float32,
                precision=lax.Precision.HIGHEST)
    feat_ref[0] = jnp.dot(f, g_ref[0], preferred_element_type=jnp.float32,
                          precision=lax.Precision.HIGHEST)
    er_ref[0] = jnp.dot(f, ar_ref[0], preferred_element_type=jnp.float32,
                        precision=lax.Precision.HIGHEST)


def _pre(xs, Ws, G, Ar16):
    nb = N // BN
    return pl.pallas_call(
        _pre_body,
        grid=(M, nb),
        in_specs=[
            pl.BlockSpec((1, BN, D_IN), lambda m, i: (m, i, 0)),
            pl.BlockSpec((1, D_IN, D), lambda m, i: (m, 0, 0)),
            pl.BlockSpec((1, D, ROW), lambda m, i: (m, 0, 0)),
            pl.BlockSpec((1, D, 16), lambda m, i: (m, 0, 0)),
        ],
        out_specs=[
            pl.BlockSpec((1, BN, ROW), lambda m, i: (m, i, 0)),
            pl.BlockSpec((1, BN, 16), lambda m, i: (m, i, 0)),
        ],
        out_shape=[
            jax.ShapeDtypeStruct((M, N, ROW), jnp.float32),
            jax.ShapeDtypeStruct((M, N, 16), jnp.float32),
        ],
    )(xs, Ws, G, Ar16)


def _edge_body(featel, er16, epad, nums,
               idx_v0, sg_v0, dg_v0, b_v0, feat_v0, out_v0,
               dstl_s0, semi0, semg0, sems0,
               idx_v1, sg_v1, dg_v1, b_v1, feat_v1, out_v1,
               dstl_s1, semi1, semg1, sems1, acc):
    c = lax.axis_index("c")
    s = lax.axis_index("s")
    lane = lax.iota(jnp.int32, 16)
    headmask = lane < H
    zero16 = jnp.zeros((16,), jnp.float32)
    bufs = ((idx_v0, sg_v0, dg_v0, b_v0, feat_v0, out_v0,
             dstl_s0, semi0, semg0, sems0),
            (idx_v1, sg_v1, dg_v1, b_v1, feat_v1, out_v1,
             dstl_s1, semi1, semg1, sems1))

    def start(m, chunk, buf):
        idx_v, sg_v, dg_v, b_v, feat_v, _, _, semi, semg, _ = buf
        base = chunk * C
        d1 = pltpu.async_copy(epad.at[m, 0, pl.ds(base, C)], idx_v.at[0],
                              semi)
        d2 = pltpu.async_copy(epad.at[m, 1, pl.ds(base, C)], idx_v.at[1],
                              semi)
        d1.wait()
        d2.wait()
        moff = m * N
        for k in range(C // 16):
            sl = pl.ds(k * 16, 16)
            sg_v[sl] = idx_v[0, sl] + moff
            dg_v[sl] = jnp.minimum(idx_v[1, sl], N - 1) + moff
        pltpu.async_copy(featel.at[sg_v], feat_v, semg)
        pltpu.async_copy(er16.at[dg_v], b_v, semg)

    def wait_gathers(buf):
        _, sg_v, dg_v, b_v, feat_v, _, _, _, semg, _ = buf
        pltpu.make_async_copy(featel.at[sg_v], feat_v, semg).wait()
        pltpu.make_async_copy(er16.at[dg_v], b_v, semg).wait()

    def wait_scatter(buf):
        _, _, _, _, _, out_v, dstl_s, _, _, sems = buf
        pltpu.make_async_copy(out_v, acc.at[dstl_s], sems).wait()

    def compute(buf):
        _, _, _, b_v, feat_v, out_v, _, _, _, _ = buf

        @plsc.parallel_loop(0, C, 1, unroll=4)
        def _edge(e):
            sv = feat_v[e, pl.ds(D, 16)] + b_v[e]
            act = jnp.maximum(sv, 0.2 * sv)
            exx = jnp.exp(act)
            out_v[e, pl.ds(D, 16)] = jnp.where(headmask, exx, 0.0)
            for g in range(H):
                out_v[e, pl.ds(g * O, O)] = feat_v[e, pl.ds(g * O, O)] * exx

    def scatter(buf):
        idx_v, _, _, _, _, out_v, dstl_s, _, _, sems = buf
        for k in range(C // 16):
            dstl_s[pl.ds(k * 16, 16)] = idx_v[1, pl.ds(k * 16, 16)]
        pltpu.async_copy(out_v, acc.at[dstl_s], sems, add=True)

    PAIRS = CHUNKS // 2
    for mi in range(2):
        m = 2 * c + mi

        def _z(i, _):
            for k in range(ROW // 16):
                out_v0[i, pl.ds(k * 16, 16)] = zero16
            return 0
        lax.fori_loop(0, C, _z, 0)
        for k in range(9):
            pltpu.sync_copy(out_v0.at[pl.ds(0, C)],
                            acc.at[pl.ds(s * (N_ACC // NS) + k * C, C)])
        pltpu.sync_copy(out_v0.at[pl.ds(0, 54)],
                        acc.at[pl.ds(s * (N_ACC // NS) + 9 * C, 54)])
        plsc.subcore_barrier()

        tile_base = s * CHUNKS
        start(m, tile_base, bufs[0])

        def _pair(i, _):
            g0 = tile_base + 2 * i
            start(m, g0 + 1, bufs[1])
            wait_gathers(bufs[0])

            @pl.when(i > 0)
            def _():
                wait_scatter(bufs[0])
            compute(bufs[0])
            scatter(bufs[0])

            @pl.when(i + 1 < PAIRS)
            def _():
                start(m, g0 + 2, bufs[0])
            wait_gathers(bufs[1])

            @pl.when(i > 0)
            def _():
                wait_scatter(bufs[1])
            compute(bufs[1])
            scatter(bufs[1])
            return 0
        lax.fori_loop(0, PAIRS, _pair, 0)
        wait_scatter(bufs[0])
        wait_scatter(bufs[1])
        plsc.subcore_barrier()

        pltpu.sync_copy(acc.at[pl.ds(s * (N_ACC // NS), N_ACC // NS)],
                        nums.at[m, pl.ds(s * (N_ACC // NS), N_ACC // NS)])
        plsc.subcore_barrier()


def _edge_phase(featel, er16, epad):
    mesh = plsc.VectorSubcoreMesh(core_axis_name="c", subcore_axis_name="s")
    buf_set = [
        pltpu.VMEM((2, C), jnp.int32),
        pltpu.VMEM((C,), jnp.int32),
        pltpu.VMEM((C,), jnp.int32),
        pltpu.VMEM((C, 16), jnp.float32),
        pltpu.VMEM((C, ROW), jnp.float32),
        pltpu.VMEM((C, ROW), jnp.float32),
        pltpu.VMEM((C,), jnp.int32),
        pltpu.SemaphoreType.DMA,
        pltpu.SemaphoreType.DMA,
        pltpu.SemaphoreType.DMA,
    ]
    return pl.kernel(
        _edge_body,
        out_type=jax.ShapeDtypeStruct((M, N_ACC, ROW), jnp.float32),
        mesh=mesh,
        scratch_types=buf_set + buf_set + [
            pltpu.VMEM_SHARED((N_ACC, ROW), jnp.float32),
        ],
        compiler_params=pltpu.CompilerParams(use_tc_tiling_on_sc=False),
    )(featel, er16, epad)


def _post_body(nums_ref, r16_ref, pt_ref, bs_ref, p1w_ref, p1b_ref,
               gat_ref, s_ref):
    nb = pl.program_id(1)
    blk = nums_ref[0]
    num = blk[:, :D]
    den = jnp.dot(blk[:, D:ROW], r16_ref[...],
                  preferred_element_type=jnp.float32,
                  precision=lax.Precision.HIGHEST)
    t = num / jnp.maximum(den, 1e-16)
    g = jnp.dot(t, pt_ref[...], preferred_element_type=jnp.float32,
                precision=lax.Precision.HIGHEST) + bs_ref[0][:1, :]
    g = jnp.maximum(g, 0.01 * g)
    gat_ref[0] = g
    th = jnp.tanh(jnp.dot(g, p1w_ref[...], preferred_element_type=jnp.float32,
                          precision=lax.Precision.HIGHEST) + p1b_ref[...])
    colsum = jnp.sum(th, axis=0, keepdims=True)

    @pl.when(nb == 0)
    def _():
        s_ref[...] = jnp.zeros_like(s_ref)
    s_ref[0] = s_ref[0] + colsum


def _post(nums, R16, Pt, bs, P1w, P1b):
    nb = N // BN
    return pl.pallas_call(
        _post_body,
        grid=(M, nb),
        in_specs=[
            pl.BlockSpec((1, BN, ROW), lambda m, i: (m, i, 0)),
            pl.BlockSpec((16, D), lambda m, i: (0, 0)),
            pl.BlockSpec((D, D), lambda m, i: (0, 0)),
            pl.BlockSpec((1, 8, D), lambda m, i: (m, 0, 0)),
            pl.BlockSpec((D, HID), lambda m, i: (0, 0)),
            pl.BlockSpec((1, HID), lambda m, i: (0, 0)),
        ],
        out_specs=[
            pl.BlockSpec((1, BN, D), lambda m, i: (m, i, 0)),
            pl.BlockSpec((1, 8, HID), lambda m, i: (m, 0, 0)),
        ],
        out_shape=[
            jax.ShapeDtypeStruct((M, N, D), jnp.float32),
            jax.ShapeDtypeStruct((M, 8, HID), jnp.float32),
        ],
    )(nums, R16, Pt, bs, P1w, P1b)


def _blend_body(gat_ref, s_ref, p2_ref, lnc_ref, dis_ref):
    w = jnp.dot(s_ref[...], p2_ref[...], preferred_element_type=jnp.float32,
                precision=lax.Precision.HIGHEST) / N

    def beta(i, j):
        wi = w[i, 0]
        wj = w[j, 0]
        mx = jnp.maximum(wi, wj)
        ei = jnp.exp(wi - mx)
        ej = jnp.exp(wj - mx)
        return ei / (ei + ej)

    b12 = beta(1, 2)
    lnc_ref[...] = b12 * gat_ref[1] + (1.0 - b12) * gat_ref[2]
    b03 = beta(0, 3)
    dis_ref[...] = b03 * gat_ref[0] + (1.0 - b03) * gat_ref[3]


def _blend(gat, s, P2w):
    nb = N // BN
    return pl.pallas_call(
        _blend_body,
        grid=(nb,),
        in_specs=[
            pl.BlockSpec((M, BN, D), lambda i: (0, i, 0)),
            pl.BlockSpec((M, HID), lambda i: (0, 0)),
            pl.BlockSpec((HID, 1), lambda i: (0, 0)),
        ],
        out_specs=[
            pl.BlockSpec((BN, D), lambda i: (i, 0)),
            pl.BlockSpec((BN, D), lambda i: (i, 0)),
        ],
        out_shape=[
            jax.ShapeDtypeStruct((N, D), jnp.float32),
            jax.ShapeDtypeStruct((N, D), jnp.float32),
        ],
    )(gat, s, P2w)


def kernel(x0, x1, x2, x3, edge_index0, edge_index1, edge_index2, edge_index3,
           W0, W1, W2, W3, al0, al1, al2, al3, ar0, ar1, ar2, ar3,
           b0, b1, b2, b3, P1w, P1b, P2w):
    xs = jnp.stack([x0, x1, x2, x3])
    Ws = jnp.stack([W0, W1, W2, W3])
    bs = jnp.stack([b0, b1, b2, b3])

    rows = jnp.arange(D)
    head_of = rows // O
    col16 = jnp.arange(16)

    def blockdiag(a):
        return jnp.where(col16[None, :] % H == head_of[:, None],
                         a.reshape(D)[:, None], 0.0)

    Al16 = jnp.stack([blockdiag(a) for a in (al0, al1, al2, al3)])
    Ar16 = jnp.stack([blockdiag(a) for a in (ar0, ar1, ar2, ar3)])
    P = ((rows % O)[:, None] * H + head_of[:, None]
         == rows[None, :]).astype(jnp.float32)
    Pt = P.T
    R16p = (col16[:, None] == (rows % H)[None, :]).astype(jnp.float32)
    G = jnp.concatenate([jnp.broadcast_to(P, (M, D, D)), Al16], axis=2)

    feat, er = _pre(xs, Ws, G, Ar16)
    featel = feat.reshape(M * N, ROW)
    er16 = er.reshape(M * N, 16)

    pad = E_PAD - E
    eis = jnp.stack([edge_index0, edge_index1, edge_index2, edge_index3])
    padc = jnp.concatenate([jnp.zeros((M, 1, pad), jnp.int32),
                            jnp.full((M, 1, pad), N, jnp.int32)], axis=1)
    epad = jnp.concatenate([eis, padc], axis=2)

    nums = _edge_phase(featel, er16, epad)

    bs8 = jnp.broadcast_to(bs[:, None, :], (M, 8, D))
    gat, s8 = _post(nums, R16p, Pt, bs8, P1w, P1b.reshape(1, HID))
    lnc, dis = _blend(gat, s8[:, 0, :], P2w)
    return (lnc, dis)
